# baseline (speedup 1.0000x reference)
import functools

import jax
import jax.numpy as jnp
from jax import lax
from jax.experimental import pallas as pl
from jax.experimental.pallas import tpu as pltpu
from jax.experimental.pallas import tpu_sc as plsc

EMBED = 64
GRP = 128


@functools.lru_cache(maxsize=None)
def _make_lookup(n_groups, embed, nc, ns, k_grps):
    nw = nc * ns
    grps_per_w = n_groups // nw
    n_chunks = grps_per_w // k_grps
    mesh = plsc.VectorSubcoreMesh(core_axis_name="c", subcore_axis_name="s")

    @functools.partial(
        pl.kernel,
        out_type=jax.ShapeDtypeStruct((n_groups, GRP, embed), jnp.float32),
        mesh=mesh,
        scratch_types=[
            pltpu.VMEM((k_grps, GRP), jnp.int32),
            pltpu.VMEM((k_grps, GRP, embed), jnp.float32),
            pltpu.SemaphoreType.DMA,
        ],
        compiler_params=pltpu.CompilerParams(use_tc_tiling_on_sc=False),
    )
    def lookup(idx_hbm, table_hbm, out_hbm, idx_v, rows_v, sem):
        wid = lax.axis_index("s") * nc + lax.axis_index("c")
        g0 = wid * grps_per_w

        def chunk(c, _):
            row = g0 + c * k_grps
            pltpu.sync_copy(idx_hbm.at[pl.ds(row, k_grps)], idx_v)
            descs = [
                pltpu.async_copy(table_hbm.at[idx_v.at[j]], rows_v.at[j], sem)
                for j in range(k_grps)
            ]
            for d in descs:
                d.wait()
            pltpu.sync_copy(rows_v, out_hbm.at[pl.ds(row, k_grps)])
            return 0

        lax.fori_loop(0, n_chunks, chunk, 0)

    return lookup


def kernel(inputs, embedding_table):
    b, s = inputs.shape
    n = b * s
    idx = inputs.reshape(n // GRP, GRP)
    info = plsc.get_sparse_core_info()
    lookup = _make_lookup(n // GRP, EMBED, info.num_cores, info.num_subcores, 4)
    out = lookup(idx, embedding_table)
    return out.reshape(b, s, EMBED)


# speedup vs baseline: 1.0443x; 1.0443x over previous
import functools

import jax
import jax.numpy as jnp
from jax import lax
from jax.experimental import pallas as pl
from jax.experimental.pallas import tpu as pltpu
from jax.experimental.pallas import tpu_sc as plsc

EMBED = 64
GRP = 128


@functools.lru_cache(maxsize=None)
def _make_lookup(n_groups, embed, nc, ns, k_grps):
    nw = nc * ns
    grps_per_w = n_groups // nw
    n_chunks = grps_per_w // k_grps
    assert grps_per_w % k_grps == 0 and n_chunks >= 4 and n_chunks % 2 == 0
    mesh = plsc.VectorSubcoreMesh(core_axis_name="c", subcore_axis_name="s")

    @functools.partial(
        pl.kernel,
        out_type=jax.ShapeDtypeStruct((n_groups, GRP, embed), jnp.float32),
        mesh=mesh,
        scratch_types=[
            pltpu.VMEM((2, k_grps, GRP), jnp.int32),
            pltpu.VMEM((2, k_grps, GRP, embed), jnp.float32),
            pltpu.SemaphoreType.DMA,
            pltpu.SemaphoreType.DMA,
            pltpu.SemaphoreType.DMA,
            pltpu.SemaphoreType.DMA,
        ],
        compiler_params=pltpu.CompilerParams(use_tc_tiling_on_sc=False),
    )
    def lookup(idx_hbm, table_hbm, out_hbm, idx_v, rows_v, sg0, sg1, sw0, sw1):
        wid = lax.axis_index("s") * nc + lax.axis_index("c")
        g0 = wid * grps_per_w
        sg = (sg0, sg1)
        sw = (sw0, sw1)

        def row_of(c):
            return g0 + c * k_grps

        def load_idx(c, p):
            pltpu.sync_copy(idx_hbm.at[pl.ds(row_of(c), k_grps)], idx_v.at[p])

        def fire_gathers(p):
            for j in range(k_grps):
                pltpu.async_copy(
                    table_hbm.at[idx_v.at[p].at[j]], rows_v.at[p].at[j], sg[p]
                )

        def drain_gathers(p):
            pltpu.make_async_copy(
                out_hbm.at[pl.ds(0, k_grps)], rows_v.at[p], sg[p]
            ).wait()

        def fire_write(c, p):
            return pltpu.async_copy(
                rows_v.at[p], out_hbm.at[pl.ds(row_of(c), k_grps)], sw[p]
            )

        load_idx(0, 0)
        fire_gathers(0)
        load_idx(1, 1)
        fire_gathers(1)

        def superstep(s, _):
            for p in range(2):
                c = 2 * s + p
                drain_gathers(p)
                w = fire_write(c, p)
                load_idx(c + 2, p)
                w.wait()
                fire_gathers(p)
            return 0

        lax.fori_loop(0, (n_chunks - 2) // 2, superstep, 0)

        for p in range(2):
            c = n_chunks - 2 + p
            drain_gathers(p)
            fire_write(c, p).wait()

    return lookup


def kernel(inputs, embedding_table):
    b, s = inputs.shape
    n = b * s
    idx = inputs.reshape(n // GRP, GRP)
    info = plsc.get_sparse_core_info()
    lookup = _make_lookup(n // GRP, EMBED, info.num_cores, info.num_subcores, 5)
    out = lookup(idx, embedding_table)
    return out.reshape(b, s, EMBED)
